# Initial kernel scaffold; baseline (speedup 1.0000x reference)
import jax
import jax.numpy as jnp
from jax.experimental import pallas as pl


def kernel(x, edge_index, edge_attr, batch, node_w, node_b, edge_w, edge_b, ln_g, ln_b, t, mlp_w1, mlp_b1, mlp_ln_g, mlp_ln_b, mlp_w2, mlp_b2, lin_w, lin_b):
    raise NotImplementedError("write your pallas kernel here")


# speedup vs baseline: 3.3102x; 3.3102x over previous
import functools

import jax
import jax.numpy as jnp
from jax import lax
from jax.experimental import pallas as pl
from jax.experimental.pallas import tpu as pltpu
from jax.experimental.pallas import tpu_sc as plsc

_N = 10000
_E = 320000
_HID = 128
_QC = 32
_NG = 64
_L = 4

_NC = 2
_NS = 16
_CH = 128
_NCHUNK = 157
_EPT = _NCHUNK * _CH
_EPAD = _NS * _EPT
_ZR = 632
_ACC_N = _NS * _ZR
_NSUB = 200
_NQ = _N // _NSUB

_BN = 400
_GN = _N // _BN
_BE = 1024
_GE = _EPAD // _BE

_f32 = jnp.float32


def _sc_kernel_deco():
    mesh = plsc.VectorSubcoreMesh(
        core_axis_name="c", subcore_axis_name="s",
        num_cores=_NC, num_subcores=_NS)
    return functools.partial(
        pl.kernel,
        out_type=jax.ShapeDtypeStruct((2 * _NC, _N, _QC), _f32),
        mesh=mesh,
        compiler_params=pltpu.CompilerParams(use_tc_tiling_on_sc=False),
        scratch_types=[
            pltpu.VMEM((_CH,), jnp.int32),
            pltpu.VMEM((_CH,), jnp.int32),
            pltpu.VMEM((_CH, _QC), _f32),
            pltpu.VMEM((_CH, _QC), _f32),
            pltpu.VMEM((_CH, _QC), _f32),
            pltpu.VMEM((_CH, _QC), _f32),
            pltpu.VMEM((16,), _f32),
            pltpu.VMEM((_NSUB, _QC), _f32),
            pltpu.VMEM((_NSUB, _QC), _f32),
            pltpu.VMEM((_NSUB, _QC), _f32),
            pltpu.SemaphoreType.DMA,
            pltpu.VMEM_SHARED((_ACC_N, _QC), _f32),
            pltpu.VMEM_SHARED((_ACC_N, _QC), _f32),
        ],
    )


def _sc_body(h0, h1, h2, h3, e0, e1, e2, e3, srcp, dstp, tvec, zrows, agg,
             src_v, dst_v, hbuf, ebuf, exbuf, mexbuf, tv,
             dbuf, nbuf, obuf, sem, acc_ex, acc_mex):
    c = lax.axis_index("c")
    s = lax.axis_index("s")

    pltpu.sync_copy(tvec, tv)
    tval = tv[...]
    z0 = s * _ZR

    def do_pass(h_t, e_t, slot):
        pltpu.sync_copy(zrows.at[pl.ds(z0, _ZR)], acc_ex.at[pl.ds(z0, _ZR)])
        pltpu.sync_copy(zrows.at[pl.ds(z0, _ZR)], acc_mex.at[pl.ds(z0, _ZR)])
        plsc.subcore_barrier()

        def chunk(i, cr):
            base = s * _EPT + i * _CH
            pltpu.sync_copy(srcp.at[pl.ds(base, _CH)], src_v)
            pltpu.sync_copy(dstp.at[pl.ds(base, _CH)], dst_v)
            pltpu.async_copy(h_t.at[src_v], hbuf, sem).wait()
            pltpu.sync_copy(e_t.at[pl.ds(base, _CH)], ebuf)

            def row(j, cr2):
                for k in range(_QC // 16):
                    sl = pl.ds(k * 16, 16)
                    m = jnp.maximum(hbuf[j, sl] + ebuf[j, sl], 0.0) + 1e-7
                    ex = jnp.exp(m * tval)
                    exbuf[j, sl] = ex
                    mexbuf[j, sl] = m * ex
                return cr2

            lax.fori_loop(0, _CH, row, 0)
            pltpu.sync_copy(exbuf, acc_ex.at[dst_v], add=True)
            pltpu.sync_copy(mexbuf, acc_mex.at[dst_v], add=True)
            return cr

        lax.fori_loop(0, _NCHUNK, chunk, 0)
        plsc.subcore_barrier()

        for qi in range(4):
            q = s + _NS * qi

            @pl.when(q < _NQ)
            def _():
                r0 = q * _NSUB
                pltpu.sync_copy(acc_ex.at[pl.ds(r0, _NSUB)], dbuf)
                pltpu.sync_copy(acc_mex.at[pl.ds(r0, _NSUB)], nbuf)

                def rowo(j, cr):
                    for k in range(_QC // 16):
                        sl = pl.ds(k * 16, 16)
                        obuf[j, sl] = nbuf[j, sl] / (dbuf[j, sl] + 1e-16)
                    return cr

                lax.fori_loop(0, _NSUB, rowo, 0)
                pltpu.sync_copy(obuf, agg.at[slot, pl.ds(r0, _NSUB)])

        plsc.subcore_barrier()

    @pl.when(c == 0)
    def _():
        do_pass(h0, e0, 0)
        do_pass(h1, e1, 1)

    @pl.when(c == 1)
    def _():
        do_pass(h2, e2, 2)
        do_pass(h3, e3, 3)


_SC_AGG = None


def _sc_softmax_agg(*args):
    global _SC_AGG
    if _SC_AGG is None:
        _SC_AGG = _sc_kernel_deco()(_sc_body)
    return _SC_AGG(*args)


def _split4(z):
    return [z[:, i * _QC:(i + 1) * _QC] for i in range(4)]


def _nodeproj_body(x_ref, w_ref, b_ref, h_ref, q0, q1, q2, q3):
    z = jnp.dot(x_ref[...], w_ref[...], preferred_element_type=_f32)
    z = z + b_ref[0:1, :]
    h_ref[...] = z
    for ref, zq in zip((q0, q1, q2, q3), _split4(z)):
        ref[...] = zq


def _edgeproj_body(a_ref, w_ref, b_ref, q0, q1, q2, q3):
    z = jnp.dot(a_ref[...], w_ref[...], preferred_element_type=_f32)
    z = z + b_ref[0:1, :]
    for ref, zq in zip((q0, q1, q2, q3), _split4(z)):
        ref[...] = zq


def _ln(z, g, b):
    mu = jnp.mean(z, axis=-1, keepdims=True)
    var = jnp.mean((z - mu) ** 2, axis=-1, keepdims=True)
    return (z - mu) * lax.rsqrt(var + 1e-5) * g + b


def _layer_body(first, a0, a1, a2, a3, r0, r1, r2, r3, hp_ref, w1_ref,
                b1_ref, g1_ref, c1_ref, w2_ref, b2_ref, gn_ref, cn_ref,
                h_ref, q0, q1, q2, q3):
    r = jnp.concatenate([r0[...], r1[...], r2[...], r3[...]], axis=1)
    a = jnp.concatenate([a0[0], a1[0], a2[0], a3[0]], axis=1)
    out = a + r
    z = jnp.dot(out, w1_ref[...], preferred_element_type=_f32) + b1_ref[0:1, :]
    z = _ln(z, g1_ref[0:1, :], c1_ref[0:1, :])
    z = jnp.maximum(z, 0.0)
    z2 = jnp.dot(z, w2_ref[...], preferred_element_type=_f32) + b2_ref[0:1, :]
    hn = z2 if first else hp_ref[...] + z2
    h_ref[...] = hn
    rn = jnp.maximum(_ln(hn, gn_ref[0:1, :], cn_ref[0:1, :]), 0.0)
    for ref, zq in zip((q0, q1, q2, q3), _split4(rn)):
        ref[...] = zq


def _head_body(r0, r1, r2, r3, b_ref, lw_ref, lb_ref, out_ref, ssum, cnt):
    i = pl.program_id(0)
    hb = jnp.concatenate([r0[...], r1[...], r2[...], r3[...]], axis=1)
    o = jnp.dot(hb, lw_ref[...], preferred_element_type=_f32) + lb_ref[0:1, :]
    brow = b_ref[0]
    gi = lax.broadcasted_iota(jnp.int32, (_NG, _BN), 0)
    oh = (gi == brow).astype(_f32)

    @pl.when(i == 0)
    def _():
        ssum[...] = jnp.zeros_like(ssum)
        cnt[...] = jnp.zeros_like(cnt)

    ssum[...] += jnp.dot(oh, o, preferred_element_type=_f32)
    cnt[...] += jnp.dot(oh, jnp.ones_like(o), preferred_element_type=_f32)

    @pl.when(i == _GN - 1)
    def _():
        out_ref[...] = ssum[...] / jnp.maximum(cnt[...], 1.0)


def _row_spec(bn, bc):
    return pl.BlockSpec((bn, bc), lambda i: (i, 0))


def _const_spec(shape):
    return pl.BlockSpec(shape, lambda i: (0, 0))


def _q_structs(n):
    return [jax.ShapeDtypeStruct((n, _QC), _f32) for _ in range(4)]


def _q_specs(bn):
    return [_row_spec(bn, _QC) for _ in range(4)]


def _agg_specs():
    mk = lambda q: pl.BlockSpec((1, _BN, _QC), lambda i, q=q: (q, i, 0))
    return [mk(q) for q in range(4)]


def _node_proj(x, w, b8):
    return pl.pallas_call(
        _nodeproj_body,
        grid=(_GN,),
        in_specs=[_row_spec(_BN, _HID), _const_spec((_HID, _HID)),
                  _const_spec((8, _HID))],
        out_specs=[_row_spec(_BN, _HID)] + _q_specs(_BN),
        out_shape=[jax.ShapeDtypeStruct((_N, _HID), _f32)] + _q_structs(_N),
    )(x, w, b8)


def _edge_proj(ea, w, b8):
    return pl.pallas_call(
        _edgeproj_body,
        grid=(_GE,),
        in_specs=[_row_spec(_BE, 16), _const_spec((16, _HID)),
                  _const_spec((8, _HID))],
        out_specs=_q_specs(_BE),
        out_shape=_q_structs(_EPAD),
    )(ea, w, b8)


def _layer_tc(first, agg, rq, hp, w1, b1, g1, c1, w2, b2, gn, cn):
    return pl.pallas_call(
        functools.partial(_layer_body, first),
        grid=(_GN,),
        in_specs=_agg_specs() + _q_specs(_BN) +
                 [_row_spec(_BN, _HID),
                  _const_spec((_HID, 2 * _HID)), _const_spec((8, 2 * _HID)),
                  _const_spec((8, 2 * _HID)), _const_spec((8, 2 * _HID)),
                  _const_spec((2 * _HID, _HID)), _const_spec((8, _HID)),
                  _const_spec((8, _HID)), _const_spec((8, _HID))],
        out_specs=[_row_spec(_BN, _HID)] + _q_specs(_BN),
        out_shape=[jax.ShapeDtypeStruct((_N, _HID), _f32)] + _q_structs(_N),
    )(agg, agg, agg, agg, *rq, hp, w1, b1, g1, c1, w2, b2, gn, cn)


def _head(rq, batch3, lwb, lb8):
    return pl.pallas_call(
        _head_body,
        grid=(_GN,),
        in_specs=_q_specs(_BN) +
                 [pl.BlockSpec((1, 1, _BN), lambda i: (i, 0, 0)),
                  _const_spec((_HID, _HID)), _const_spec((8, _HID))],
        out_specs=pl.BlockSpec((_NG, _HID), lambda i: (0, 0)),
        out_shape=jax.ShapeDtypeStruct((_NG, _HID), _f32),
        scratch_shapes=[pltpu.VMEM((_NG, _HID), _f32),
                        pltpu.VMEM((_NG, _HID), _f32)],
    )(*rq, batch3, lwb, lb8)


def _tile8(v):
    return jnp.tile(v.reshape(1, -1), (8, 1))


def kernel(x, edge_index, edge_attr, batch, node_w, node_b, edge_w, edge_b,
           ln_g, ln_b, t, mlp_w1, mlp_b1, mlp_ln_g, mlp_ln_b, mlp_w2, mlp_b2,
           lin_w, lin_b):
    pad = _EPAD - _E
    src = jnp.concatenate([edge_index[0], jnp.zeros((pad,), jnp.int32)])
    dst = jnp.concatenate([edge_index[1], jnp.full((pad,), _N, jnp.int32)])
    eap = jnp.concatenate([edge_attr, jnp.zeros((pad, 16), _f32)])
    tb = jnp.tile(t.reshape(_L, 1), (1, 16)).astype(_f32)
    zrows = jnp.zeros((_ACC_N, _QC), _f32)
    batch3 = batch.reshape(_GN, 1, _BN)

    h, *rq = _node_proj(x, node_w, _tile8(node_b))
    eq = _edge_proj(eap, edge_w, _tile8(edge_b))

    for i in range(_L):
        agg = _sc_softmax_agg(*rq, *eq, src, dst, tb[i], zrows)
        gn = ln_g[(i + 1) % _L]
        cn = ln_b[(i + 1) % _L]
        h, *rq = _layer_tc(
            i == 0, agg, rq, h,
            mlp_w1[i], _tile8(mlp_b1[i]), _tile8(mlp_ln_g[i]),
            _tile8(mlp_ln_b[i]), mlp_w2[i], _tile8(mlp_b2[i]),
            _tile8(gn), _tile8(cn))

    lwb = jnp.tile(lin_w, (1, _HID))
    out = _head(rq, batch3, lwb, _tile8(lin_b))
    return out[:, :1]


# speedup vs baseline: 4.8685x; 1.4708x over previous
import functools

import jax
import jax.numpy as jnp
from jax import lax
from jax.experimental import pallas as pl
from jax.experimental.pallas import tpu as pltpu
from jax.experimental.pallas import tpu_sc as plsc

_N = 10000
_E = 320000
_HID = 128
_QC = 16
_NSL = 8
_NG = 64
_L = 4

_NC = 2
_NS = 16
_CH = 128
_NCHUNK = 158
_EPT = _NCHUNK * _CH
_EPAD = _NS * _EPT
_ZR = 632
_ACC_N = _NS * _ZR
_NSUB = 200
_NQ = _N // _NSUB

_BN = 400
_GN = _N // _BN
_BE = 1024
_GE = _EPAD // _BE

_f32 = jnp.float32


def _sc_kernel_deco():
    mesh = plsc.VectorSubcoreMesh(
        core_axis_name="c", subcore_axis_name="s",
        num_cores=_NC, num_subcores=_NS)
    return functools.partial(
        pl.kernel,
        out_type=jax.ShapeDtypeStruct((_NSL, _N, _QC), _f32),
        mesh=mesh,
        compiler_params=pltpu.CompilerParams(use_tc_tiling_on_sc=False),
        scratch_types=[
            pltpu.VMEM((_NCHUNK, _CH), jnp.int32),
            pltpu.VMEM((_NCHUNK, _CH), jnp.int32),
            pltpu.VMEM((_CH, _QC), _f32),
            pltpu.VMEM((_CH, _QC), _f32),
            pltpu.VMEM((_CH, _QC), _f32),
            pltpu.VMEM((_CH, _QC), _f32),
            pltpu.VMEM((_CH, _QC), _f32),
            pltpu.VMEM((_CH, _QC), _f32),
            pltpu.VMEM((_CH, _QC), _f32),
            pltpu.VMEM((_CH, _QC), _f32),
            pltpu.VMEM((16,), _f32),
            pltpu.VMEM((_NSUB, _QC), _f32),
            pltpu.VMEM((_NSUB, _QC), _f32),
            pltpu.VMEM((_NSUB, _QC), _f32),
            pltpu.SemaphoreType.DMA,
            pltpu.SemaphoreType.DMA,
            pltpu.SemaphoreType.DMA,
            pltpu.SemaphoreType.DMA,
            pltpu.SemaphoreType.DMA,
            pltpu.SemaphoreType.DMA,
            pltpu.VMEM_SHARED((_ACC_N, _QC), _f32),
            pltpu.VMEM_SHARED((_ACC_N, _QC), _f32),
        ],
    )


def _sc_body(h0, h1, h2, h3, h4, h5, h6, h7,
             e0, e1, e2, e3, e4, e5, e6, e7,
             srcp, dstp, tvec, zrows, agg,
             src2d, dst2d, hbufA, hbufB, ebufA, ebufB, exA, mexA, exB, mexB,
             tv, dbuf, nbuf, obuf, sgA, sgB, seA, seB, ssA, ssB,
             acc_ex, acc_mex):
    c = lax.axis_index("c")
    s = lax.axis_index("s")

    pltpu.sync_copy(tvec, tv)
    pltpu.sync_copy(srcp.at[pl.ds(s * _NCHUNK, _NCHUNK)], src2d)
    pltpu.sync_copy(dstp.at[pl.ds(s * _NCHUNK, _NCHUNK)], dst2d)
    tval = tv[...]
    z0 = s * _ZR

    def do_pass(h_t, e_t, slot):
        pltpu.sync_copy(zrows.at[pl.ds(z0, _ZR)], acc_ex.at[pl.ds(z0, _ZR)])
        pltpu.sync_copy(zrows.at[pl.ds(z0, _ZR)], acc_mex.at[pl.ds(z0, _ZR)])
        plsc.subcore_barrier()

        ebase = s * _EPT

        def compute(hb, eb, exb, mexb):
            def row(j, cr2):
                m = jnp.maximum(hb[j] + eb[j], 0.0) + 1e-7
                ex = jnp.exp(m * tval)
                exb[j] = ex
                mexb[j] = m * ex
                return cr2

            lax.fori_loop(0, _CH, row, 0)

        def halfstep(a, hb, eb, exb, mexb, sg, se, ss, hb2, eb2, sg2, se2):
            @pl.when(a + 1 < _NCHUNK)
            def _():
                pltpu.async_copy(h_t.at[src2d.at[a + 1]], hb2, sg2)
                pltpu.async_copy(
                    e_t.at[pl.ds(ebase + (a + 1) * _CH, _CH)], eb2, se2)

            pltpu.make_async_copy(h_t.at[src2d.at[a]], hb, sg).wait()
            pltpu.make_async_copy(
                e_t.at[pl.ds(ebase + a * _CH, _CH)], eb, se).wait()

            @pl.when(a >= 2)
            def _():
                pltpu.make_async_copy(exb, acc_ex.at[dst2d.at[a]], ss).wait()
                pltpu.make_async_copy(mexb, acc_mex.at[dst2d.at[a]], ss).wait()

            compute(hb, eb, exb, mexb)
            pltpu.async_copy(exb, acc_ex.at[dst2d.at[a]], ss, add=True)
            pltpu.async_copy(mexb, acc_mex.at[dst2d.at[a]], ss, add=True)

        pltpu.async_copy(h_t.at[src2d.at[0]], hbufA, sgA)
        pltpu.async_copy(e_t.at[pl.ds(ebase, _CH)], ebufA, seA)

        def body(it, cr):
            a = 2 * it
            halfstep(a, hbufA, ebufA, exA, mexA, sgA, seA, ssA,
                     hbufB, ebufB, sgB, seB)
            halfstep(a + 1, hbufB, ebufB, exB, mexB, sgB, seB, ssB,
                     hbufA, ebufA, sgA, seA)
            return cr

        lax.fori_loop(0, _NCHUNK // 2, body, 0)

        pltpu.make_async_copy(exA, acc_ex.at[dst2d.at[0]], ssA).wait()
        pltpu.make_async_copy(mexA, acc_mex.at[dst2d.at[0]], ssA).wait()
        pltpu.make_async_copy(exB, acc_ex.at[dst2d.at[0]], ssB).wait()
        pltpu.make_async_copy(mexB, acc_mex.at[dst2d.at[0]], ssB).wait()
        plsc.subcore_barrier()

        for qi in range(4):
            q = s + _NS * qi

            @pl.when(q < _NQ)
            def _():
                r0 = q * _NSUB
                pltpu.sync_copy(acc_ex.at[pl.ds(r0, _NSUB)], dbuf)
                pltpu.sync_copy(acc_mex.at[pl.ds(r0, _NSUB)], nbuf)

                def rowo(j, cr):
                    obuf[j] = nbuf[j] / (dbuf[j] + 1e-16)
                    return cr

                lax.fori_loop(0, _NSUB, rowo, 0)
                pltpu.sync_copy(obuf, agg.at[slot, pl.ds(r0, _NSUB)])

        plsc.subcore_barrier()

    hs = (h0, h1, h2, h3, h4, h5, h6, h7)
    es = (e0, e1, e2, e3, e4, e5, e6, e7)

    @pl.when(c == 0)
    def _():
        for p in range(4):
            do_pass(hs[p], es[p], p)

    @pl.when(c == 1)
    def _():
        for p in range(4):
            do_pass(hs[4 + p], es[4 + p], 4 + p)


_SC_AGG = None


def _sc_softmax_agg(*args):
    global _SC_AGG
    if _SC_AGG is None:
        _SC_AGG = _sc_kernel_deco()(_sc_body)
    return _SC_AGG(*args)


def _split8(z):
    return [z[:, i * _QC:(i + 1) * _QC] for i in range(_NSL)]


def _nodeproj_body(x_ref, w_ref, b_ref, h_ref, *qs):
    z = jnp.dot(x_ref[...], w_ref[...], preferred_element_type=_f32)
    z = z + b_ref[0:1, :]
    h_ref[...] = z
    for ref, zq in zip(qs, _split8(z)):
        ref[...] = zq


def _edgeproj_body(a_ref, w_ref, b_ref, *qs):
    z = jnp.dot(a_ref[...], w_ref[...], preferred_element_type=_f32)
    z = z + b_ref[0:1, :]
    for ref, zq in zip(qs, _split8(z)):
        ref[...] = zq


def _ln(z, g, b):
    mu = jnp.mean(z, axis=-1, keepdims=True)
    var = jnp.mean((z - mu) ** 2, axis=-1, keepdims=True)
    return (z - mu) * lax.rsqrt(var + 1e-5) * g + b


def _layer_body(first, *refs):
    aggs = refs[:_NSL]
    rs = refs[_NSL:2 * _NSL]
    (hp_ref, w1_ref, b1_ref, g1_ref, c1_ref, w2_ref, b2_ref, gn_ref,
     cn_ref) = refs[2 * _NSL:2 * _NSL + 9]
    h_ref = refs[2 * _NSL + 9]
    qs = refs[2 * _NSL + 10:]
    r = jnp.concatenate([rr[...] for rr in rs], axis=1)
    a = jnp.concatenate([aa[0] for aa in aggs], axis=1)
    out = a + r
    z = jnp.dot(out, w1_ref[...], preferred_element_type=_f32) + b1_ref[0:1, :]
    z = _ln(z, g1_ref[0:1, :], c1_ref[0:1, :])
    z = jnp.maximum(z, 0.0)
    z2 = jnp.dot(z, w2_ref[...], preferred_element_type=_f32) + b2_ref[0:1, :]
    hn = z2 if first else hp_ref[...] + z2
    h_ref[...] = hn
    rn = jnp.maximum(_ln(hn, gn_ref[0:1, :], cn_ref[0:1, :]), 0.0)
    for ref, zq in zip(qs, _split8(rn)):
        ref[...] = zq


def _head_body(*refs):
    rs = refs[:_NSL]
    b_ref, lw_ref, lb_ref, out_ref, ssum, cnt = refs[_NSL:]
    i = pl.program_id(0)
    hb = jnp.concatenate([rr[...] for rr in rs], axis=1)
    o = jnp.dot(hb, lw_ref[...], preferred_element_type=_f32) + lb_ref[0:1, :]
    brow = b_ref[0]
    gi = lax.broadcasted_iota(jnp.int32, (_NG, _BN), 0)
    oh = (gi == brow).astype(_f32)

    @pl.when(i == 0)
    def _():
        ssum[...] = jnp.zeros_like(ssum)
        cnt[...] = jnp.zeros_like(cnt)

    ssum[...] += jnp.dot(oh, o, preferred_element_type=_f32)
    cnt[...] += jnp.dot(oh, jnp.ones_like(o), preferred_element_type=_f32)

    @pl.when(i == _GN - 1)
    def _():
        out_ref[...] = ssum[...] / jnp.maximum(cnt[...], 1.0)


def _row_spec(bn, bc):
    return pl.BlockSpec((bn, bc), lambda i: (i, 0))


def _const_spec(shape):
    return pl.BlockSpec(shape, lambda i: (0, 0))


def _q_structs(n):
    return [jax.ShapeDtypeStruct((n, _QC), _f32) for _ in range(_NSL)]


def _q_specs(bn):
    return [_row_spec(bn, _QC) for _ in range(_NSL)]


def _agg_specs():
    mk = lambda q: pl.BlockSpec((1, _BN, _QC), lambda i, q=q: (q, i, 0))
    return [mk(q) for q in range(_NSL)]


def _node_proj(x, w, b8):
    return pl.pallas_call(
        _nodeproj_body,
        grid=(_GN,),
        in_specs=[_row_spec(_BN, _HID), _const_spec((_HID, _HID)),
                  _const_spec((8, _HID))],
        out_specs=[_row_spec(_BN, _HID)] + _q_specs(_BN),
        out_shape=[jax.ShapeDtypeStruct((_N, _HID), _f32)] + _q_structs(_N),
    )(x, w, b8)


def _edge_proj(ea, w, b8):
    return pl.pallas_call(
        _edgeproj_body,
        grid=(_GE,),
        in_specs=[_row_spec(_BE, 16), _const_spec((16, _HID)),
                  _const_spec((8, _HID))],
        out_specs=_q_specs(_BE),
        out_shape=_q_structs(_EPAD),
    )(ea, w, b8)


def _layer_tc(first, agg, rq, hp, w1, b1, g1, c1, w2, b2, gn, cn):
    return pl.pallas_call(
        functools.partial(_layer_body, first),
        grid=(_GN,),
        in_specs=_agg_specs() + _q_specs(_BN) +
                 [_row_spec(_BN, _HID),
                  _const_spec((_HID, 2 * _HID)), _const_spec((8, 2 * _HID)),
                  _const_spec((8, 2 * _HID)), _const_spec((8, 2 * _HID)),
                  _const_spec((2 * _HID, _HID)), _const_spec((8, _HID)),
                  _const_spec((8, _HID)), _const_spec((8, _HID))],
        out_specs=[_row_spec(_BN, _HID)] + _q_specs(_BN),
        out_shape=[jax.ShapeDtypeStruct((_N, _HID), _f32)] + _q_structs(_N),
    )(*([agg] * _NSL), *rq, hp, w1, b1, g1, c1, w2, b2, gn, cn)


def _head(rq, batch3, lwb, lb8):
    return pl.pallas_call(
        _head_body,
        grid=(_GN,),
        in_specs=_q_specs(_BN) +
                 [pl.BlockSpec((1, 1, _BN), lambda i: (i, 0, 0)),
                  _const_spec((_HID, _HID)), _const_spec((8, _HID))],
        out_specs=pl.BlockSpec((_NG, _HID), lambda i: (0, 0)),
        out_shape=jax.ShapeDtypeStruct((_NG, _HID), _f32),
        scratch_shapes=[pltpu.VMEM((_NG, _HID), _f32),
                        pltpu.VMEM((_NG, _HID), _f32)],
    )(*rq, batch3, lwb, lb8)


def _tile8(v):
    return jnp.tile(v.reshape(1, -1), (8, 1))


def kernel(x, edge_index, edge_attr, batch, node_w, node_b, edge_w, edge_b,
           ln_g, ln_b, t, mlp_w1, mlp_b1, mlp_ln_g, mlp_ln_b, mlp_w2, mlp_b2,
           lin_w, lin_b):
    pad = _EPAD - _E
    src = jnp.concatenate(
        [edge_index[0], jnp.zeros((pad,), jnp.int32)]).reshape(
            _NS * _NCHUNK, _CH)
    dst = jnp.concatenate(
        [edge_index[1], jnp.full((pad,), _N, jnp.int32)]).reshape(
            _NS * _NCHUNK, _CH)
    eap = jnp.concatenate([edge_attr, jnp.zeros((pad, 16), _f32)])
    tb = jnp.tile(t.reshape(_L, 1), (1, 16)).astype(_f32)
    zrows = jnp.zeros((_ACC_N, _QC), _f32)
    batch3 = batch.reshape(_GN, 1, _BN)

    h, *rq = _node_proj(x, node_w, _tile8(node_b))
    eq = _edge_proj(eap, edge_w, _tile8(edge_b))

    for i in range(_L):
        agg = _sc_softmax_agg(*rq, *eq, src, dst, tb[i], zrows)
        gn = ln_g[(i + 1) % _L]
        cn = ln_b[(i + 1) % _L]
        h, *rq = _layer_tc(
            i == 0, agg, rq, h,
            mlp_w1[i], _tile8(mlp_b1[i]), _tile8(mlp_ln_g[i]),
            _tile8(mlp_ln_b[i]), mlp_w2[i], _tile8(mlp_b2[i]),
            _tile8(gn), _tile8(cn))

    lwb = jnp.tile(lin_w, (1, _HID))
    out = _head(rq, batch3, lwb, _tile8(lin_b))
    return out[:, :1]


# speedup vs baseline: 5.2240x; 1.0730x over previous
import functools

import jax
import jax.numpy as jnp
from jax import lax
from jax.experimental import pallas as pl
from jax.experimental.pallas import tpu as pltpu
from jax.experimental.pallas import tpu_sc as plsc

_N = 10000
_E = 320000
_HID = 128
_QC = 16
_NSL = 8
_NG = 64
_L = 4

_NC = 2
_NS = 16
_CH = 128
_NCHUNK = 158
_EPT = _NCHUNK * _CH
_EPAD = _NS * _EPT
_ZR = 632
_ACC_N = _NS * _ZR
_NSUB = 200
_NQ = _N // _NSUB

_BN = 400
_GN = _N // _BN
_BE = 1024
_GE = _EPAD // _BE

_f32 = jnp.float32


def _sc_kernel_deco():
    mesh = plsc.VectorSubcoreMesh(
        core_axis_name="c", subcore_axis_name="s",
        num_cores=_NC, num_subcores=_NS)
    return functools.partial(
        pl.kernel,
        out_type=jax.ShapeDtypeStruct((_NSL, _N, _QC), _f32),
        mesh=mesh,
        compiler_params=pltpu.CompilerParams(use_tc_tiling_on_sc=False),
        scratch_types=[
            pltpu.VMEM((_NCHUNK, _CH), jnp.int32),
            pltpu.VMEM((_NCHUNK, _CH), jnp.int32),
            pltpu.VMEM((_CH, _QC), _f32),
            pltpu.VMEM((_CH, _QC), _f32),
            pltpu.VMEM((_CH, _QC), _f32),
            pltpu.VMEM((_CH, _QC), _f32),
            pltpu.VMEM((_CH, _QC), _f32),
            pltpu.VMEM((_CH, _QC), _f32),
            pltpu.VMEM((_CH, _QC), _f32),
            pltpu.VMEM((_CH, _QC), _f32),
            pltpu.VMEM((16,), _f32),
            pltpu.VMEM((_NSUB, _QC), _f32),
            pltpu.VMEM((_NSUB, _QC), _f32),
            pltpu.VMEM((_NSUB, _QC), _f32),
            pltpu.SemaphoreType.DMA,
            pltpu.SemaphoreType.DMA,
            pltpu.SemaphoreType.DMA,
            pltpu.SemaphoreType.DMA,
            pltpu.SemaphoreType.DMA,
            pltpu.SemaphoreType.DMA,
            pltpu.VMEM_SHARED((_ACC_N, _QC), _f32),
            pltpu.VMEM_SHARED((_ACC_N, _QC), _f32),
        ],
    )


def _sc_body(h0, h1, h2, h3, h4, h5, h6, h7,
             e0, e1, e2, e3, e4, e5, e6, e7,
             srcp, dstp, tvec, zrows, agg,
             src2d, dst2d, hbufA, hbufB, ebufA, ebufB, exA, mexA, exB, mexB,
             tv, dbuf, nbuf, obuf, sgA, sgB, seA, seB, ssA, ssB,
             acc_ex, acc_mex):
    c = lax.axis_index("c")
    s = lax.axis_index("s")

    pltpu.sync_copy(tvec, tv)
    pltpu.sync_copy(srcp.at[pl.ds(s * _NCHUNK, _NCHUNK)], src2d)
    pltpu.sync_copy(dstp.at[pl.ds(s * _NCHUNK, _NCHUNK)], dst2d)
    tval = tv[...]
    z0 = s * _ZR

    def do_pass(h_t, e_t, slot):
        pltpu.sync_copy(zrows.at[pl.ds(z0, _ZR)], acc_ex.at[pl.ds(z0, _ZR)])
        pltpu.sync_copy(zrows.at[pl.ds(z0, _ZR)], acc_mex.at[pl.ds(z0, _ZR)])
        plsc.subcore_barrier()

        ebase = s * _EPT

        def compute(hb, eb, exb, mexb):
            def row(i, cr2):
                j0 = i * 8
                for k in range(8):
                    j = j0 + k
                    m = jnp.maximum(hb[j] + eb[j], 0.0) + 1e-7
                    ex = jnp.exp(m * tval)
                    exb[j] = ex
                    mexb[j] = m * ex
                return cr2

            lax.fori_loop(0, _CH // 8, row, 0)

        def halfstep(a, hb, eb, exb, mexb, sg, se, ss, hb2, eb2, sg2, se2):
            @pl.when(a + 1 < _NCHUNK)
            def _():
                pltpu.async_copy(h_t.at[src2d.at[a + 1]], hb2, sg2)
                pltpu.async_copy(
                    e_t.at[pl.ds(ebase + (a + 1) * _CH, _CH)], eb2, se2)

            pltpu.make_async_copy(h_t.at[src2d.at[a]], hb, sg).wait()
            pltpu.make_async_copy(
                e_t.at[pl.ds(ebase + a * _CH, _CH)], eb, se).wait()

            @pl.when(a >= 2)
            def _():
                pltpu.make_async_copy(exb, acc_ex.at[dst2d.at[a]], ss).wait()
                pltpu.make_async_copy(mexb, acc_mex.at[dst2d.at[a]], ss).wait()

            compute(hb, eb, exb, mexb)
            pltpu.async_copy(exb, acc_ex.at[dst2d.at[a]], ss, add=True)
            pltpu.async_copy(mexb, acc_mex.at[dst2d.at[a]], ss, add=True)

        pltpu.async_copy(h_t.at[src2d.at[0]], hbufA, sgA)
        pltpu.async_copy(e_t.at[pl.ds(ebase, _CH)], ebufA, seA)

        def body(it, cr):
            a = 2 * it
            halfstep(a, hbufA, ebufA, exA, mexA, sgA, seA, ssA,
                     hbufB, ebufB, sgB, seB)
            halfstep(a + 1, hbufB, ebufB, exB, mexB, sgB, seB, ssB,
                     hbufA, ebufA, sgA, seA)
            return cr

        lax.fori_loop(0, _NCHUNK // 2, body, 0)

        pltpu.make_async_copy(exA, acc_ex.at[dst2d.at[0]], ssA).wait()
        pltpu.make_async_copy(mexA, acc_mex.at[dst2d.at[0]], ssA).wait()
        pltpu.make_async_copy(exB, acc_ex.at[dst2d.at[0]], ssB).wait()
        pltpu.make_async_copy(mexB, acc_mex.at[dst2d.at[0]], ssB).wait()
        plsc.subcore_barrier()

        for qi in range(4):
            q = s + _NS * qi

            @pl.when(q < _NQ)
            def _():
                r0 = q * _NSUB
                pltpu.sync_copy(acc_ex.at[pl.ds(r0, _NSUB)], dbuf)
                pltpu.sync_copy(acc_mex.at[pl.ds(r0, _NSUB)], nbuf)

                def rowo(i, cr):
                    j0 = i * 8
                    for k in range(8):
                        j = j0 + k
                        obuf[j] = nbuf[j] / (dbuf[j] + 1e-16)
                    return cr

                lax.fori_loop(0, _NSUB // 8, rowo, 0)
                pltpu.sync_copy(obuf, agg.at[slot, pl.ds(r0, _NSUB)])

        plsc.subcore_barrier()

    hs = (h0, h1, h2, h3, h4, h5, h6, h7)
    es = (e0, e1, e2, e3, e4, e5, e6, e7)

    @pl.when(c == 0)
    def _():
        for p in range(4):
            do_pass(hs[p], es[p], p)

    @pl.when(c == 1)
    def _():
        for p in range(4):
            do_pass(hs[4 + p], es[4 + p], 4 + p)


_SC_AGG = None


def _sc_softmax_agg(*args):
    global _SC_AGG
    if _SC_AGG is None:
        _SC_AGG = _sc_kernel_deco()(_sc_body)
    return _SC_AGG(*args)


def _split8(z):
    return [z[:, i * _QC:(i + 1) * _QC] for i in range(_NSL)]


def _nodeproj_body(x_ref, w_ref, b_ref, h_ref, *qs):
    z = jnp.dot(x_ref[...], w_ref[...], preferred_element_type=_f32)
    z = z + b_ref[0:1, :]
    h_ref[...] = z
    for ref, zq in zip(qs, _split8(z)):
        ref[...] = zq


def _edgeproj_body(a_ref, w_ref, b_ref, *qs):
    z = jnp.dot(a_ref[...], w_ref[...], preferred_element_type=_f32)
    z = z + b_ref[0:1, :]
    for ref, zq in zip(qs, _split8(z)):
        ref[...] = zq


def _ln(z, g, b):
    mu = jnp.mean(z, axis=-1, keepdims=True)
    var = jnp.mean((z - mu) ** 2, axis=-1, keepdims=True)
    return (z - mu) * lax.rsqrt(var + 1e-5) * g + b


def _layer_body(first, *refs):
    aggs = refs[:_NSL]
    rs = refs[_NSL:2 * _NSL]
    (hp_ref, w1_ref, b1_ref, g1_ref, c1_ref, w2_ref, b2_ref, gn_ref,
     cn_ref) = refs[2 * _NSL:2 * _NSL + 9]
    h_ref = refs[2 * _NSL + 9]
    qs = refs[2 * _NSL + 10:]
    r = jnp.concatenate([rr[...] for rr in rs], axis=1)
    a = jnp.concatenate([aa[0] for aa in aggs], axis=1)
    out = a + r
    z = jnp.dot(out, w1_ref[...], preferred_element_type=_f32) + b1_ref[0:1, :]
    z = _ln(z, g1_ref[0:1, :], c1_ref[0:1, :])
    z = jnp.maximum(z, 0.0)
    z2 = jnp.dot(z, w2_ref[...], preferred_element_type=_f32) + b2_ref[0:1, :]
    hn = z2 if first else hp_ref[...] + z2
    h_ref[...] = hn
    rn = jnp.maximum(_ln(hn, gn_ref[0:1, :], cn_ref[0:1, :]), 0.0)
    for ref, zq in zip(qs, _split8(rn)):
        ref[...] = zq


def _head_body(*refs):
    rs = refs[:_NSL]
    b_ref, lw_ref, lb_ref, out_ref, ssum, cnt = refs[_NSL:]
    i = pl.program_id(0)
    hb = jnp.concatenate([rr[...] for rr in rs], axis=1)
    o = jnp.dot(hb, lw_ref[...], preferred_element_type=_f32) + lb_ref[0:1, :]
    brow = b_ref[0]
    gi = lax.broadcasted_iota(jnp.int32, (_NG, _BN), 0)
    oh = (gi == brow).astype(_f32)

    @pl.when(i == 0)
    def _():
        ssum[...] = jnp.zeros_like(ssum)
        cnt[...] = jnp.zeros_like(cnt)

    ssum[...] += jnp.dot(oh, o, preferred_element_type=_f32)
    cnt[...] += jnp.dot(oh, jnp.ones_like(o), preferred_element_type=_f32)

    @pl.when(i == _GN - 1)
    def _():
        out_ref[...] = ssum[...] / jnp.maximum(cnt[...], 1.0)


def _row_spec(bn, bc):
    return pl.BlockSpec((bn, bc), lambda i: (i, 0))


def _const_spec(shape):
    return pl.BlockSpec(shape, lambda i: (0, 0))


def _q_structs(n):
    return [jax.ShapeDtypeStruct((n, _QC), _f32) for _ in range(_NSL)]


def _q_specs(bn):
    return [_row_spec(bn, _QC) for _ in range(_NSL)]


def _agg_specs():
    mk = lambda q: pl.BlockSpec((1, _BN, _QC), lambda i, q=q: (q, i, 0))
    return [mk(q) for q in range(_NSL)]


def _node_proj(x, w, b8):
    return pl.pallas_call(
        _nodeproj_body,
        grid=(_GN,),
        in_specs=[_row_spec(_BN, _HID), _const_spec((_HID, _HID)),
                  _const_spec((8, _HID))],
        out_specs=[_row_spec(_BN, _HID)] + _q_specs(_BN),
        out_shape=[jax.ShapeDtypeStruct((_N, _HID), _f32)] + _q_structs(_N),
    )(x, w, b8)


def _edge_proj(ea, w, b8):
    return pl.pallas_call(
        _edgeproj_body,
        grid=(_GE,),
        in_specs=[_row_spec(_BE, 16), _const_spec((16, _HID)),
                  _const_spec((8, _HID))],
        out_specs=_q_specs(_BE),
        out_shape=_q_structs(_EPAD),
    )(ea, w, b8)


def _layer_tc(first, agg, rq, hp, w1, b1, g1, c1, w2, b2, gn, cn):
    return pl.pallas_call(
        functools.partial(_layer_body, first),
        grid=(_GN,),
        in_specs=_agg_specs() + _q_specs(_BN) +
                 [_row_spec(_BN, _HID),
                  _const_spec((_HID, 2 * _HID)), _const_spec((8, 2 * _HID)),
                  _const_spec((8, 2 * _HID)), _const_spec((8, 2 * _HID)),
                  _const_spec((2 * _HID, _HID)), _const_spec((8, _HID)),
                  _const_spec((8, _HID)), _const_spec((8, _HID))],
        out_specs=[_row_spec(_BN, _HID)] + _q_specs(_BN),
        out_shape=[jax.ShapeDtypeStruct((_N, _HID), _f32)] + _q_structs(_N),
    )(*([agg] * _NSL), *rq, hp, w1, b1, g1, c1, w2, b2, gn, cn)


def _head(rq, batch3, lwb, lb8):
    return pl.pallas_call(
        _head_body,
        grid=(_GN,),
        in_specs=_q_specs(_BN) +
                 [pl.BlockSpec((1, 1, _BN), lambda i: (i, 0, 0)),
                  _const_spec((_HID, _HID)), _const_spec((8, _HID))],
        out_specs=pl.BlockSpec((_NG, _HID), lambda i: (0, 0)),
        out_shape=jax.ShapeDtypeStruct((_NG, _HID), _f32),
        scratch_shapes=[pltpu.VMEM((_NG, _HID), _f32),
                        pltpu.VMEM((_NG, _HID), _f32)],
    )(*rq, batch3, lwb, lb8)


def _tile8(v):
    return jnp.tile(v.reshape(1, -1), (8, 1))


def kernel(x, edge_index, edge_attr, batch, node_w, node_b, edge_w, edge_b,
           ln_g, ln_b, t, mlp_w1, mlp_b1, mlp_ln_g, mlp_ln_b, mlp_w2, mlp_b2,
           lin_w, lin_b):
    pad = _EPAD - _E
    src = jnp.concatenate(
        [edge_index[0], jnp.zeros((pad,), jnp.int32)]).reshape(
            _NS * _NCHUNK, _CH)
    dst = jnp.concatenate(
        [edge_index[1], jnp.full((pad,), _N, jnp.int32)]).reshape(
            _NS * _NCHUNK, _CH)
    eap = jnp.concatenate([edge_attr, jnp.zeros((pad, 16), _f32)])
    tb = jnp.tile(t.reshape(_L, 1), (1, 16)).astype(_f32)
    zrows = jnp.zeros((_ACC_N, _QC), _f32)
    batch3 = batch.reshape(_GN, 1, _BN)

    h, *rq = _node_proj(x, node_w, _tile8(node_b))
    eq = _edge_proj(eap, edge_w, _tile8(edge_b))

    for i in range(_L):
        agg = _sc_softmax_agg(*rq, *eq, src, dst, tb[i], zrows)
        gn = ln_g[(i + 1) % _L]
        cn = ln_b[(i + 1) % _L]
        h, *rq = _layer_tc(
            i == 0, agg, rq, h,
            mlp_w1[i], _tile8(mlp_b1[i]), _tile8(mlp_ln_g[i]),
            _tile8(mlp_ln_b[i]), mlp_w2[i], _tile8(mlp_b2[i]),
            _tile8(gn), _tile8(cn))

    lwb = jnp.tile(lin_w, (1, _HID))
    out = _head(rq, batch3, lwb, _tile8(lin_b))
    return out[:, :1]
